# baseline (speedup 1.0000x reference)
import functools

import jax
import jax.numpy as jnp
from jax import lax
from jax.experimental import pallas as pl
from jax.experimental.pallas import tpu as pltpu
from jax.experimental.pallas import tpu_sc as plsc

N = 10000
E = 320000
D = 128
H = 128
T = 4
G = 256

NC = 2
NS = 16
NW = NC * NS

EW = E // NW
SUB = 80
CH = 80
SPC = CH // SUB
NCH = EW // CH
NPAD = 10240
ROWS_T = NPAD // NS
EXPAD = 40960
EXT = EXPAD // NS

_f32 = jnp.float32
_i32 = jnp.int32


def _sc_body(hp, ssr, sds, pidx, agg_out, ex_out,
             ssv, sdv, rows0, rows1, eidx0, eidx1, scat0, scat1, exv0, exv1,
             zex_v, agg_sh, ex_sh, sem_i, sem_g, sem_s):
    cid = lax.axis_index("c")
    sid = lax.axis_index("s")
    wid = cid * NS + sid

    rows = (rows0, rows1)
    eidx = (eidx0, eidx1)
    scat = (scat0, scat1)
    exvv = (exv0, exv1)

    z16f = jnp.zeros((16,), _f32)
    for r in range(CH):
        for c in range(8):
            rows0[r, pl.ds(c * 16, 16)] = z16f
    for i in range(EXT // 16):
        zex_v[pl.ds(i * 16, 16)] = z16f
    for r in range(ROWS_T // CH):
        pltpu.sync_copy(rows0, agg_sh.at[pl.ds(sid * ROWS_T + r * CH, CH)])
    pltpu.sync_copy(zex_v, ex_sh.at[pl.ds(sid * EXT, EXT)])

    pltpu.sync_copy(ssr, ssv)
    pltpu.sync_copy(sds, sdv)
    plsc.subcore_barrier()

    def process(c, b, drain_prev=True, start_next=True):
        bo = 1 - b
        pltpu.make_async_copy(pidx.at[wid * NCH + c], eidx[b], sem_i).wait()
        g = pltpu.async_copy(hp.at[eidx[b].at[0]], rows[b], sem_g)
        for i in range(SUB // 16):
            sl = pl.ds(i * 16, 16)
            sv = eidx[b][0, sl]
            dv = eidx[b][1, sl]
            tv = eidx[b][2, sl]
            a = plsc.load_gather(ssv, [sv >> 7, sv & 127])
            bb = plsc.load_gather(sdv, [dv >> 7, dv & 127])
            s = a + bb
            s = jnp.where(s >= 0.0, s, 0.2 * s)
            exv = jnp.exp(s)
            exvv[b][sl] = exv
            scat[b][sl] = dv * T + tv
        if drain_prev:
            pltpu.make_async_copy(rows[bo],
                                  agg_sh.at[eidx[bo].at[1]], sem_s).wait()
            pltpu.make_async_copy(exvv[bo],
                                  ex_sh.at[scat[bo]], sem_s).wait()
        if start_next:
            pltpu.async_copy(pidx.at[wid * NCH + c + 1], eidx[bo], sem_i)
        g.wait()

        rb = rows[b]
        eb = exvv[b]

        @plsc.parallel_loop(0, SUB, 1, unroll=4)
        def srow(jj):
            exs = plsc.load_gather(eb, [jnp.zeros((16,), _i32) + jj])
            for cc in range(H // 16):
                sl = pl.ds(cc * 16, 16)
                rb[jj, sl] = rb[jj, sl] * exs

        pltpu.async_copy(rows[b], agg_sh.at[eidx[b].at[1]], sem_s, add=True)
        pltpu.async_copy(exvv[b], ex_sh.at[scat[b]], sem_s, add=True)

    pltpu.async_copy(pidx.at[wid * NCH], eidx[0], sem_i)
    process(0, 0, drain_prev=False)
    process(1, 1)

    def pair(p, carry):
        c = 2 * p
        process(c, 0)
        process(c + 1, 1)
        return carry

    lax.fori_loop(1, NCH // 2, pair, 0)
    process(NCH - 1, 0, start_next=False)
    pltpu.make_async_copy(rows0, agg_sh.at[eidx0.at[1]], sem_s).wait()
    pltpu.make_async_copy(exv0, ex_sh.at[scat0], sem_s).wait()
    plsc.subcore_barrier()

    pltpu.sync_copy(agg_sh.at[pl.ds(sid * ROWS_T, ROWS_T)],
                    agg_out.at[cid, pl.ds(sid * ROWS_T, ROWS_T)])
    pltpu.sync_copy(ex_sh.at[pl.ds(sid * EXT, EXT)],
                    ex_out.at[cid, pl.ds(sid * EXT, EXT)])


_sc_layer = functools.partial(
    pl.kernel,
    out_type=[
        jax.ShapeDtypeStruct((NC, NPAD, H), _f32),
        jax.ShapeDtypeStruct((NC, EXPAD), _f32),
    ],
    mesh=plsc.VectorSubcoreMesh(core_axis_name="c", subcore_axis_name="s"),
    compiler_params=pltpu.CompilerParams(needs_layout_passes=False),
    scratch_types=[
        pltpu.VMEM((NPAD // 128, 128), _f32),
        pltpu.VMEM((NPAD // 128, 128), _f32),
        pltpu.VMEM((CH, H), _f32),
        pltpu.VMEM((CH, H), _f32),
        pltpu.VMEM((3, SUB), _i32),
        pltpu.VMEM((3, SUB), _i32),
        pltpu.VMEM((SUB,), _i32),
        pltpu.VMEM((SUB,), _i32),
        pltpu.VMEM((SUB,), _f32),
        pltpu.VMEM((SUB,), _f32),
        pltpu.VMEM((EXT,), _f32),
        pltpu.VMEM_SHARED((NPAD, H), _f32),
        pltpu.VMEM_SHARED((EXPAD,), _f32),
        pltpu.SemaphoreType.DMA,
        pltpu.SemaphoreType.DMA,
        pltpu.SemaphoreType.DMA,
    ],
)(_sc_body)


def _tc0_body(x_ref, w_ref, asrc_ref, adst_ref, hp_ref, ssr_ref, sds_ref):
    hp = jnp.dot(x_ref[...], w_ref[...], preferred_element_type=_f32, precision=lax.Precision.HIGHEST)
    hp_ref[...] = hp
    ssr_ref[...] = jnp.sum(hp * asrc_ref[...][None, :], axis=1)
    sds_ref[...] = jnp.sum(hp * adst_ref[...][None, :], axis=1)


_tc0 = pl.pallas_call(
    _tc0_body,
    out_shape=[
        jax.ShapeDtypeStruct((N, H), _f32),
        jax.ShapeDtypeStruct((N,), _f32),
        jax.ShapeDtypeStruct((N,), _f32),
    ],
)


def _combine_h(aggp_ref, ex3_ref, emb_ref):
    ex = ex3_ref[0] + ex3_ref[1]
    agg = aggp_ref[0, :N] + aggp_ref[1, :N] + jnp.dot(
        ex, emb_ref[...], preferred_element_type=_f32, precision=lax.Precision.HIGHEST)
    denom = jnp.sum(ex, axis=1) + 1e-16
    return jnp.maximum(agg / denom[:, None], 0.0)


def _comb_body(aggp_ref, ex3_ref, emb_ref, w_ref, asrc_ref, adst_ref,
               hp_ref, ssr_ref, sds_ref):
    h = _combine_h(aggp_ref, ex3_ref, emb_ref)
    hp = jnp.dot(h, w_ref[...], preferred_element_type=_f32, precision=lax.Precision.HIGHEST)
    hp_ref[...] = hp
    ssr_ref[...] = jnp.sum(hp * asrc_ref[...][None, :], axis=1)
    sds_ref[...] = jnp.sum(hp * adst_ref[...][None, :], axis=1)


_tc_comb = pl.pallas_call(
    _comb_body,
    out_shape=[
        jax.ShapeDtypeStruct((N, H), _f32),
        jax.ShapeDtypeStruct((N,), _f32),
        jax.ShapeDtypeStruct((N,), _f32),
    ],
)


def _final_body(aggp_ref, ex3_ref, emb_ref, batch_ref, encw_ref, encb_ref,
                fw_ref, fb_ref, out_ref):
    h = _combine_h(aggp_ref, ex3_ref, emb_ref)
    bt = batch_ref[...]
    gi = lax.broadcasted_iota(_i32, (G, N), 0)
    onehot = (bt[None, :] == gi).astype(_f32)
    counts = jnp.sum(onehot, axis=1)
    xg = jnp.dot(onehot, h, preferred_element_type=_f32, precision=lax.Precision.HIGHEST)
    xg = xg / jnp.clip(counts, 1.0, None)[:, None]
    xg = jnp.maximum(
        jnp.dot(xg, encw_ref[0], preferred_element_type=_f32, precision=lax.Precision.HIGHEST)
        + encb_ref[0][None, :], 0.0)
    xg = jnp.maximum(
        jnp.dot(xg, encw_ref[1], preferred_element_type=_f32, precision=lax.Precision.HIGHEST)
        + encb_ref[1][None, :], 0.0)
    out_ref[...] = jnp.sum(xg * fw_ref[...][None, :], axis=1) + fb_ref[0]


_tc_final = pl.pallas_call(
    _final_body,
    out_shape=jax.ShapeDtypeStruct((G,), _f32),
)


def kernel(x, edge_index, edge_type, batch, gnn_W, a_src, a_dst, edge_emb,
           enc_W, enc_b, final_W, final_b):
    esrc = edge_index[0].astype(_i32)
    edst = edge_index[1].astype(_i32)
    etyp = edge_type.astype(_i32)
    pidx = (jnp.stack([esrc, edst, etyp])
            .reshape(3, NW, NCH, SUB)
            .transpose(1, 2, 0, 3)
            .reshape(NW * NCH, 3, SUB))
    hp, ssr, sds = _tc0(x, gnn_W[0], a_src[0], a_dst[0])
    out = None
    for l in range(gnn_W.shape[0]):
        ssr2 = jnp.pad(ssr, (0, NPAD - N)).reshape(NPAD // 128, 128)
        sds2 = jnp.pad(sds, (0, NPAD - N)).reshape(NPAD // 128, 128)
        aggp, exo = _sc_layer(hp, ssr2, sds2, pidx)
        ex3 = exo[:, :N * T].reshape(NC, N, T)
        if l + 1 < gnn_W.shape[0]:
            hp, ssr, sds = _tc_comb(aggp, ex3, edge_emb[l], gnn_W[l + 1],
                                    a_src[l + 1], a_dst[l + 1])
        else:
            out = _tc_final(aggp, ex3, edge_emb[l], batch.astype(_i32),
                            enc_W, enc_b, final_W,
                            final_b.reshape(1).astype(_f32))
    return out


# speedup vs baseline: 1.1512x; 1.1512x over previous
import functools

import jax
import jax.numpy as jnp
from jax import lax
from jax.experimental import pallas as pl
from jax.experimental.pallas import tpu as pltpu
from jax.experimental.pallas import tpu_sc as plsc

N = 10000
E = 320000
D = 128
H = 128
T = 4
G = 256

NC = 2
NS = 16
NW = NC * NS

EW = E // NW
SUB = 80
CH = 80
SPC = CH // SUB
NCH = EW // CH
NPAD = 10240
ROWS_T = NPAD // NS
EXPAD = 40960
EXT = EXPAD // NS

_f32 = jnp.float32
_i32 = jnp.int32


def _sc_body(hp, ssr, sds, pidx, agg_out, ex_out,
             ssv, sdv, rows0, rows1, eidx0, eidx1, eidx2,
             scat0, scat1, scat2, exv0, exv1, exv2,
             zex_v, agg_sh, ex_sh, sem_i, sem_g, sem_s):
    cid = lax.axis_index("c")
    sid = lax.axis_index("s")
    wid = cid * NS + sid

    rows = (rows0, rows1)
    eidx = (eidx0, eidx1, eidx2)
    scat = (scat0, scat1, scat2)
    exvv = (exv0, exv1, exv2)

    z16f = jnp.zeros((16,), _f32)
    for r in range(CH):
        for c in range(8):
            rows0[r, pl.ds(c * 16, 16)] = z16f
    for i in range(EXT // 16):
        zex_v[pl.ds(i * 16, 16)] = z16f
    for r in range(ROWS_T // CH):
        pltpu.sync_copy(rows0, agg_sh.at[pl.ds(sid * ROWS_T + r * CH, CH)])
    pltpu.sync_copy(zex_v, ex_sh.at[pl.ds(sid * EXT, EXT)])

    pltpu.sync_copy(ssr, ssv)
    pltpu.sync_copy(sds, sdv)
    plsc.subcore_barrier()

    def idx_fire(c, t):
        pltpu.async_copy(pidx.at[wid * NCH + c], eidx[t], sem_i)

    def body(c, b, t, drain_prev=True, fire_idx=True, fire_next=True):
        bo = 1 - b
        tp = (t + 2) % 3
        tn = (t + 1) % 3
        pltpu.make_async_copy(hp.at[eidx[t].at[0]], rows[b], sem_g).wait()
        if drain_prev:
            pltpu.make_async_copy(rows[bo],
                                  agg_sh.at[eidx[tp].at[1]], sem_s).wait()
            pltpu.make_async_copy(exvv[tp], ex_sh.at[scat[tp]], sem_s).wait()
        if fire_idx:
            idx_fire(c + 2, tp)
        for i in range(SUB // 16):
            sl = pl.ds(i * 16, 16)
            sv = eidx[t][0, sl]
            dv = eidx[t][1, sl]
            tv = eidx[t][2, sl]
            a = plsc.load_gather(ssv, [sv >> 7, sv & 127])
            bb = plsc.load_gather(sdv, [dv >> 7, dv & 127])
            s = a + bb
            s = jnp.where(s >= 0.0, s, 0.2 * s)
            exv = jnp.exp(s)
            exvv[t][sl] = exv
            scat[t][sl] = dv * T + tv
        if fire_next:
            pltpu.make_async_copy(pidx.at[wid * NCH + c + 1], eidx[tn],
                                  sem_i).wait()
            pltpu.async_copy(hp.at[eidx[tn].at[0]], rows[bo], sem_g)

        rb = rows[b]
        eb = exvv[t]

        @plsc.parallel_loop(0, SUB, 1, unroll=2)
        def srow(jj):
            exs = plsc.load_gather(eb, [jnp.zeros((16,), _i32) + jj])
            for cc in range(H // 16):
                sl = pl.ds(cc * 16, 16)
                rb[jj, sl] = rb[jj, sl] * exs

        pltpu.async_copy(rows[b], agg_sh.at[eidx[t].at[1]], sem_s, add=True)
        pltpu.async_copy(exvv[t], ex_sh.at[scat[t]], sem_s, add=True)

    idx_fire(0, 0)
    idx_fire(1, 1)
    pltpu.make_async_copy(pidx.at[wid * NCH], eidx[0], sem_i).wait()
    pltpu.async_copy(hp.at[eidx[0].at[0]], rows[0], sem_g)
    body(0, 0, 0, drain_prev=False)
    body(1, 1, 1)

    def six(p, carry):
        c = 2 + 6 * p
        for k in range(6):
            body(c + k, k % 2, (2 + k) % 3)
        return carry

    lax.fori_loop(0, (NCH - 5) // 6, six, 0)
    body(NCH - 3, 0, (NCH - 3) % 3)
    body(NCH - 2, 1, (NCH - 2) % 3, fire_idx=False)
    body(NCH - 1, 0, (NCH - 1) % 3, fire_idx=False, fire_next=False)
    tl = (NCH - 1) % 3
    pltpu.make_async_copy(rows[0], agg_sh.at[eidx[tl].at[1]], sem_s).wait()
    pltpu.make_async_copy(exvv[tl], ex_sh.at[scat[tl]], sem_s).wait()
    plsc.subcore_barrier()

    pltpu.sync_copy(agg_sh.at[pl.ds(sid * ROWS_T, ROWS_T)],
                    agg_out.at[cid, pl.ds(sid * ROWS_T, ROWS_T)])
    pltpu.sync_copy(ex_sh.at[pl.ds(sid * EXT, EXT)],
                    ex_out.at[cid, pl.ds(sid * EXT, EXT)])


_sc_layer = functools.partial(
    pl.kernel,
    out_type=[
        jax.ShapeDtypeStruct((NC, NPAD, H), _f32),
        jax.ShapeDtypeStruct((NC, EXPAD), _f32),
    ],
    mesh=plsc.VectorSubcoreMesh(core_axis_name="c", subcore_axis_name="s"),
    compiler_params=pltpu.CompilerParams(needs_layout_passes=False),
    scratch_types=[
        pltpu.VMEM((NPAD // 128, 128), _f32),
        pltpu.VMEM((NPAD // 128, 128), _f32),
        pltpu.VMEM((CH, H), _f32),
        pltpu.VMEM((CH, H), _f32),
        pltpu.VMEM((3, SUB), _i32),
        pltpu.VMEM((3, SUB), _i32),
        pltpu.VMEM((3, SUB), _i32),
        pltpu.VMEM((SUB,), _i32),
        pltpu.VMEM((SUB,), _i32),
        pltpu.VMEM((SUB,), _i32),
        pltpu.VMEM((SUB,), _f32),
        pltpu.VMEM((SUB,), _f32),
        pltpu.VMEM((SUB,), _f32),
        pltpu.VMEM((EXT,), _f32),
        pltpu.VMEM_SHARED((NPAD, H), _f32),
        pltpu.VMEM_SHARED((EXPAD,), _f32),
        pltpu.SemaphoreType.DMA,
        pltpu.SemaphoreType.DMA,
        pltpu.SemaphoreType.DMA,
    ],
)(_sc_body)


def _tc0_body(x_ref, w_ref, asrc_ref, adst_ref, hp_ref, ssr_ref, sds_ref):
    hp = jnp.dot(x_ref[...], w_ref[...], preferred_element_type=_f32, precision=lax.Precision.HIGHEST)
    hp_ref[...] = hp
    ssr_ref[...] = jnp.sum(hp * asrc_ref[...][None, :], axis=1)
    sds_ref[...] = jnp.sum(hp * adst_ref[...][None, :], axis=1)


_tc0 = pl.pallas_call(
    _tc0_body,
    out_shape=[
        jax.ShapeDtypeStruct((N, H), _f32),
        jax.ShapeDtypeStruct((N,), _f32),
        jax.ShapeDtypeStruct((N,), _f32),
    ],
)


def _combine_h(aggp_ref, ex3_ref, emb_ref):
    ex = ex3_ref[0] + ex3_ref[1]
    agg = aggp_ref[0, :N] + aggp_ref[1, :N] + jnp.dot(
        ex, emb_ref[...], preferred_element_type=_f32, precision=lax.Precision.HIGHEST)
    denom = jnp.sum(ex, axis=1) + 1e-16
    return jnp.maximum(agg / denom[:, None], 0.0)


def _comb_body(aggp_ref, ex3_ref, emb_ref, w_ref, asrc_ref, adst_ref,
               hp_ref, ssr_ref, sds_ref):
    h = _combine_h(aggp_ref, ex3_ref, emb_ref)
    hp = jnp.dot(h, w_ref[...], preferred_element_type=_f32, precision=lax.Precision.HIGHEST)
    hp_ref[...] = hp
    ssr_ref[...] = jnp.sum(hp * asrc_ref[...][None, :], axis=1)
    sds_ref[...] = jnp.sum(hp * adst_ref[...][None, :], axis=1)


_tc_comb = pl.pallas_call(
    _comb_body,
    out_shape=[
        jax.ShapeDtypeStruct((N, H), _f32),
        jax.ShapeDtypeStruct((N,), _f32),
        jax.ShapeDtypeStruct((N,), _f32),
    ],
)


def _final_body(aggp_ref, ex3_ref, emb_ref, batch_ref, encw_ref, encb_ref,
                fw_ref, fb_ref, out_ref):
    h = _combine_h(aggp_ref, ex3_ref, emb_ref)
    bt = batch_ref[...]
    gi = lax.broadcasted_iota(_i32, (G, N), 0)
    onehot = (bt[None, :] == gi).astype(_f32)
    counts = jnp.sum(onehot, axis=1)
    xg = jnp.dot(onehot, h, preferred_element_type=_f32, precision=lax.Precision.HIGHEST)
    xg = xg / jnp.clip(counts, 1.0, None)[:, None]
    xg = jnp.maximum(
        jnp.dot(xg, encw_ref[0], preferred_element_type=_f32, precision=lax.Precision.HIGHEST)
        + encb_ref[0][None, :], 0.0)
    xg = jnp.maximum(
        jnp.dot(xg, encw_ref[1], preferred_element_type=_f32, precision=lax.Precision.HIGHEST)
        + encb_ref[1][None, :], 0.0)
    out_ref[...] = jnp.sum(xg * fw_ref[...][None, :], axis=1) + fb_ref[0]


_tc_final = pl.pallas_call(
    _final_body,
    out_shape=jax.ShapeDtypeStruct((G,), _f32),
)


def kernel(x, edge_index, edge_type, batch, gnn_W, a_src, a_dst, edge_emb,
           enc_W, enc_b, final_W, final_b):
    esrc = edge_index[0].astype(_i32)
    edst = edge_index[1].astype(_i32)
    etyp = edge_type.astype(_i32)
    pidx = (jnp.stack([esrc, edst, etyp])
            .reshape(3, NW, NCH, SUB)
            .transpose(1, 2, 0, 3)
            .reshape(NW * NCH, 3, SUB))
    hp, ssr, sds = _tc0(x, gnn_W[0], a_src[0], a_dst[0])
    out = None
    for l in range(gnn_W.shape[0]):
        ssr2 = jnp.pad(ssr, (0, NPAD - N)).reshape(NPAD // 128, 128)
        sds2 = jnp.pad(sds, (0, NPAD - N)).reshape(NPAD // 128, 128)
        aggp, exo = _sc_layer(hp, ssr2, sds2, pidx)
        ex3 = exo[:, :N * T].reshape(NC, N, T)
        if l + 1 < gnn_W.shape[0]:
            hp, ssr, sds = _tc_comb(aggp, ex3, edge_emb[l], gnn_W[l + 1],
                                    a_src[l + 1], a_dst[l + 1])
        else:
            out = _tc_final(aggp, ex3, edge_emb[l], batch.astype(_i32),
                            enc_W, enc_b, final_W,
                            final_b.reshape(1).astype(_f32))
    return out
